# baseline (speedup 1.0000x reference)
import functools

import jax
import jax.numpy as jnp
from jax import lax
from jax.experimental import pallas as pl
from jax.experimental.pallas import tpu as pltpu
from jax.experimental.pallas import tpu_sc as plsc

_NC = 2
_NS = 16
_NW = _NC * _NS
_LANES = 16
_BLK = 48
_NBUF = 3


def _body(x_hbm, pe_hbm, out_hbm, b0, b1, b2, pe_v, ld_sem, st_sem,
          *, L, D, seg, nblk_w):
    bufs = (b0, b1, b2)
    cid = lax.axis_index("c")
    sid = lax.axis_index("s")
    wid = sid * _NC + cid
    nd = D // _LANES
    nwords = _BLK * D
    bpj = seg // _BLK
    jobs_w = nblk_w // bpj
    nsuper = nblk_w // _NBUF

    pltpu.sync_copy(pe_hbm.at[pl.ds(0, 3 * D)], pe_v)

    def w0_off(g):
        job = wid * jobs_w + g // bpj
        jb = lax.rem(g, bpj)
        bq = job // 2
        s = lax.rem(job, 2)
        row0 = bq * L + 1 + s * seg + jb * _BLK
        return row0 * D, (1 + s) * D

    def start_load(g, k):
        w0, _ = w0_off(g)
        pltpu.async_copy(x_hbm.at[pl.ds(w0, nwords)], bufs[k], ld_sem.at[k])

    def wait_load(g, k):
        w0, _ = w0_off(g)
        pltpu.make_async_copy(
            x_hbm.at[pl.ds(w0, nwords)], bufs[k], ld_sem.at[k]).wait()

    def start_store(g, k):
        w0, _ = w0_off(g)
        pltpu.async_copy(bufs[k], out_hbm.at[pl.ds(w0, nwords)], st_sem.at[k])

    def wait_store(g, k):
        w0, _ = w0_off(g)
        pltpu.make_async_copy(
            bufs[k], out_hbm.at[pl.ds(w0, nwords)], st_sem.at[k]).wait()

    def compute(k, off):
        buf = bufs[k]

        def dloop(d, c):
            pev = pe_v[pl.ds(off + d * _LANES, _LANES)]

            def rloop(r, c2):
                sl = pl.ds(r * D + d * _LANES, _LANES)
                buf[sl] = buf[sl] + pev
                return c2

            return lax.fori_loop(0, _BLK, rloop, c, unroll=8)

        lax.fori_loop(0, nd, dloop, 0)

    start_load(0, 0)

    def superstep(step, carry):
        for k in range(_NBUF):
            g = step * _NBUF + k
            kn = (k + 1) % _NBUF
            if k < _NBUF - 1:
                @pl.when(step >= 1)
                def _():
                    wait_store(g - 2, kn)
                start_load(g + 1, kn)
            else:
                @pl.when(step < nsuper - 1)
                def _():
                    wait_store(g - 2, kn)
                    start_load(g + 1, kn)
            wait_load(g, k)
            _, off = w0_off(g)
            compute(k, off)
            start_store(g, k)
        return carry

    lax.fori_loop(0, nsuper, superstep, 0)
    for k in range(_NBUF):
        wait_store(nblk_w - _NBUF + k, k)

    def cls_block(j, carry):
        w0 = (wid + j * _NW) * L * D
        pltpu.sync_copy(x_hbm.at[pl.ds(w0, D)], b0.at[pl.ds(0, D)])
        for d in range(nd):
            sl = pl.ds(d * _LANES, _LANES)
            b0[sl] = b0[sl] + pe_v[pl.ds(d * _LANES, _LANES)]
        pltpu.sync_copy(b0.at[pl.ds(0, D)], out_hbm.at[pl.ds(w0, D)])
        return carry

    lax.fori_loop(0, 2, cls_block, 0)


def kernel(x, pe):
    B, L, D = x.shape
    seg = (L - 1) // 2
    rows = B * L
    nblk_w = (rows - B) // (_NW * _BLK)

    body = functools.partial(_body, L=L, D=D, seg=seg, nblk_w=nblk_w)
    mesh = plsc.VectorSubcoreMesh(
        core_axis_name="c", subcore_axis_name="s",
        num_cores=_NC, num_subcores=_NS)
    out = pl.kernel(
        body,
        out_type=jax.ShapeDtypeStruct((rows * D,), jnp.float32),
        mesh=mesh,
        scratch_types=[
            pltpu.VMEM((_BLK * D,), jnp.float32),
            pltpu.VMEM((_BLK * D,), jnp.float32),
            pltpu.VMEM((_BLK * D,), jnp.float32),
            pltpu.VMEM((3 * D,), jnp.float32),
            pltpu.SemaphoreType.DMA((_NBUF,)),
            pltpu.SemaphoreType.DMA((_NBUF,)),
        ],
    )(x.reshape(-1), pe.reshape(-1))
    return out.reshape(x.shape)


# speedup vs baseline: 1.4475x; 1.4475x over previous
import functools

import jax
import jax.numpy as jnp
from jax import lax
from jax.experimental import pallas as pl
from jax.experimental.pallas import tpu as pltpu
from jax.experimental.pallas import tpu_sc as plsc

_NC = 2
_NS = 16
_NW = _NC * _NS
_LANES = 16
_BLK = 48
_NBUF = 3


def _body(x_hbm, pe_hbm, out_hbm, b0, b1, b2, pe_v, ld_sem, st_sem,
          *, L, D, seg, nblk_w):
    bufs = (b0, b1, b2)
    cid = lax.axis_index("c")
    sid = lax.axis_index("s")
    wid = sid * _NC + cid
    nd = D // _LANES
    nwords = _BLK * D
    bpj = seg // _BLK
    jobs_w = nblk_w // bpj
    nsuper = nblk_w // _NBUF

    pltpu.sync_copy(pe_hbm.at[pl.ds(0, 3 * D)], pe_v)

    def w0_off(g):
        job = wid * jobs_w + g // bpj
        jb = lax.rem(g, bpj)
        bq = job // 2
        s = lax.rem(job, 2)
        row0 = bq * L + 1 + s * seg + jb * _BLK
        return row0 * D, (1 + s) * D

    def start_load(g, k):
        w0, _ = w0_off(g)
        pltpu.async_copy(x_hbm.at[pl.ds(w0, nwords)], bufs[k], ld_sem.at[k])

    def wait_load(g, k):
        w0, _ = w0_off(g)
        pltpu.make_async_copy(
            x_hbm.at[pl.ds(w0, nwords)], bufs[k], ld_sem.at[k]).wait()

    def start_store(g, k):
        w0, _ = w0_off(g)
        pltpu.async_copy(bufs[k], out_hbm.at[pl.ds(w0, nwords)], st_sem.at[k])

    def wait_store(g, k):
        w0, _ = w0_off(g)
        pltpu.make_async_copy(
            bufs[k], out_hbm.at[pl.ds(w0, nwords)], st_sem.at[k]).wait()

    def compute(k, off):
        buf = bufs[k]
        for d in range(nd):
            pev = pe_v[pl.ds(off + d * _LANES, _LANES)]

            @plsc.parallel_loop(0, _BLK, unroll=8)
            def _(r, _d=d, _pev=pev):
                sl = pl.ds(r * D + _d * _LANES, _LANES)
                buf[sl] = buf[sl] + _pev

    start_load(0, 0)

    def superstep(step, carry):
        for k in range(_NBUF):
            g = step * _NBUF + k
            kn = (k + 1) % _NBUF
            if k < _NBUF - 1:
                @pl.when(step >= 1)
                def _():
                    wait_store(g - 2, kn)
                start_load(g + 1, kn)
            else:
                @pl.when(step < nsuper - 1)
                def _():
                    wait_store(g - 2, kn)
                    start_load(g + 1, kn)
            wait_load(g, k)
            _, off = w0_off(g)
            compute(k, off)
            start_store(g, k)
        return carry

    lax.fori_loop(0, nsuper, superstep, 0)
    for k in range(_NBUF):
        wait_store(nblk_w - _NBUF + k, k)

    def cls_block(j, carry):
        w0 = (wid + j * _NW) * L * D
        pltpu.sync_copy(x_hbm.at[pl.ds(w0, D)], b0.at[pl.ds(0, D)])
        for d in range(nd):
            sl = pl.ds(d * _LANES, _LANES)
            b0[sl] = b0[sl] + pe_v[pl.ds(d * _LANES, _LANES)]
        pltpu.sync_copy(b0.at[pl.ds(0, D)], out_hbm.at[pl.ds(w0, D)])
        return carry

    lax.fori_loop(0, 2, cls_block, 0)


def kernel(x, pe):
    B, L, D = x.shape
    seg = (L - 1) // 2
    rows = B * L
    nblk_w = (rows - B) // (_NW * _BLK)

    body = functools.partial(_body, L=L, D=D, seg=seg, nblk_w=nblk_w)
    mesh = plsc.VectorSubcoreMesh(
        core_axis_name="c", subcore_axis_name="s",
        num_cores=_NC, num_subcores=_NS)
    out = pl.kernel(
        body,
        out_type=jax.ShapeDtypeStruct((rows * D,), jnp.float32),
        mesh=mesh,
        scratch_types=[
            pltpu.VMEM((_BLK * D,), jnp.float32),
            pltpu.VMEM((_BLK * D,), jnp.float32),
            pltpu.VMEM((_BLK * D,), jnp.float32),
            pltpu.VMEM((3 * D,), jnp.float32),
            pltpu.SemaphoreType.DMA((_NBUF,)),
            pltpu.SemaphoreType.DMA((_NBUF,)),
        ],
    )(x.reshape(-1), pe.reshape(-1))
    return out.reshape(x.shape)


# speedup vs baseline: 2.8787x; 1.9887x over previous
import functools

import jax
import jax.numpy as jnp
from jax import lax
from jax.experimental import pallas as pl
from jax.experimental.pallas import tpu as pltpu
from jax.experimental.pallas import tpu_sc as plsc

_NC = 2
_NS = 16
_NW = _NC * _NS
_LANES = 16
_BLK = 48
_NBUF = 3


def _body(x_hbm, pe_hbm, out_hbm, b0, b1, b2, tail, pe_v, cor_v, ld_sem,
          st_sem, *, B, L, D, seg):
    bufs = (b0, b1, b2)
    cid = lax.axis_index("c")
    sid = lax.axis_index("s")
    wid = sid * _NC + cid
    nd = D // _LANES
    kpj = (L - 1) // _BLK
    kmid = (seg // _BLK)
    nblk_w = B * kpj // _NW
    nsuper = nblk_w // _NBUF

    for r in range(3):
        pltpu.sync_copy(pe_hbm.at[r, :], pe_v.at[pl.ds(r * D, D)])
    for d in range(nd):
        s0 = pl.ds(d * _LANES, _LANES)
        s1 = pl.ds(D + d * _LANES, _LANES)
        s2 = pl.ds(2 * D + d * _LANES, _LANES)
        cor_v[s0] = pe_v[s0] - pe_v[s1]
        cor_v[s1] = pe_v[s1] - pe_v[s2]

    def bk(g):
        t = wid * nblk_w + g
        return t // kpj, lax.rem(t, kpj)

    def start_load(g, k):
        b, kk = bk(g)
        pltpu.async_copy(
            x_hbm.at[b, pl.ds(kk * _BLK, _BLK), :], bufs[k], ld_sem.at[k])

    def wait_load(g, k):
        b, kk = bk(g)
        pltpu.make_async_copy(
            x_hbm.at[b, pl.ds(kk * _BLK, _BLK), :], bufs[k],
            ld_sem.at[k]).wait()

    def start_store(g, k):
        b, kk = bk(g)
        pltpu.async_copy(
            bufs[k], out_hbm.at[b, pl.ds(kk * _BLK, _BLK), :], st_sem.at[k])

    def wait_store(g, k):
        b, kk = bk(g)
        pltpu.make_async_copy(
            bufs[k], out_hbm.at[b, pl.ds(kk * _BLK, _BLK), :],
            st_sem.at[k]).wait()

    def compute(g, k):
        buf = bufs[k]
        _, kk = bk(g)
        off = jnp.where(kk <= kmid - 1, D, 2 * D)
        for d in range(nd):
            pev = pe_v[pl.ds(off + d * _LANES, _LANES)]

            @plsc.parallel_loop(0, _BLK, unroll=8)
            def _(r, _d=d, _pev=pev):
                sl = pl.ds(_d * _LANES, _LANES)
                buf[r, sl] = buf[r, sl] + _pev

        @pl.when(jnp.logical_or(kk == 0, kk == kmid))
        def _():
            coff = jnp.where(kk == 0, 0, D)
            for d in range(nd):
                sl = pl.ds(d * _LANES, _LANES)
                buf[0, sl] = buf[0, sl] + cor_v[pl.ds(coff + d * _LANES,
                                                      _LANES)]

    start_load(0, 0)

    def superstep(step, carry):
        for k in range(_NBUF):
            g = step * _NBUF + k
            kn = (k + 1) % _NBUF
            if k < _NBUF - 1:
                @pl.when(step >= 1)
                def _():
                    wait_store(g - 2, kn)
                start_load(g + 1, kn)
            else:
                @pl.when(step < nsuper - 1)
                def _():
                    wait_store(g - 2, kn)
                    start_load(g + 1, kn)
            wait_load(g, k)
            compute(g, k)
            start_store(g, k)
        return carry

    lax.fori_loop(0, nsuper, superstep, 0)
    for k in range(_NBUF):
        wait_store(nblk_w - _NBUF + k, k)

    def tail_block(j, carry):
        b = wid + j * _NW
        pltpu.sync_copy(x_hbm.at[b, pl.ds(L - 1, 1), :], tail)
        for d in range(nd):
            sl = pl.ds(d * _LANES, _LANES)
            tail[0, sl] = tail[0, sl] + pe_v[pl.ds(2 * D + d * _LANES,
                                                   _LANES)]
        pltpu.sync_copy(tail, out_hbm.at[b, pl.ds(L - 1, 1), :])
        return carry

    lax.fori_loop(0, B // _NW, tail_block, 0)


def kernel(x, pe):
    B, L, D = x.shape
    seg = (L - 1) // 2

    body = functools.partial(_body, B=B, L=L, D=D, seg=seg)
    mesh = plsc.VectorSubcoreMesh(
        core_axis_name="c", subcore_axis_name="s",
        num_cores=_NC, num_subcores=_NS)
    return pl.kernel(
        body,
        out_type=jax.ShapeDtypeStruct((B, L, D), jnp.float32),
        mesh=mesh,
        scratch_types=[
            pltpu.VMEM((_BLK, D), jnp.float32),
            pltpu.VMEM((_BLK, D), jnp.float32),
            pltpu.VMEM((_BLK, D), jnp.float32),
            pltpu.VMEM((1, D), jnp.float32),
            pltpu.VMEM((3 * D,), jnp.float32),
            pltpu.VMEM((2 * D,), jnp.float32),
            pltpu.SemaphoreType.DMA((_NBUF,)),
            pltpu.SemaphoreType.DMA((_NBUF,)),
        ],
    )(x, pe)
